# baseline (speedup 1.0000x reference)
import functools

import jax
import jax.numpy as jnp
from jax.experimental import pallas as pl
from jax.experimental.pallas import tpu as pltpu
from jax.experimental.pallas import tpu_sc as plsc

K = 16


def _mlp(x, layers):
    n = len(layers)
    for i, (W, b) in enumerate(layers):
        x = x @ W + b
        if i < n - 1:
            x = jax.nn.relu(x)
    return x


def _gather(x, idx):
    return jax.vmap(lambda xb, ib: xb[ib])(x, idx)


def _knn_kernel(q_ref, s_ref, o_ref, *, n_s, k):
    q = q_ref[0]
    s = s_ref[0]
    q2 = jnp.sum(q * q, axis=1)
    s2 = jnp.sum(s * s, axis=1)
    qs = jax.lax.dot_general(s, q, (((1,), (1,)), ((), ())),
                             preferred_element_type=jnp.float32)
    d2 = s2[:, None] + q2[None, :] - 2.0 * qs
    iota = jax.lax.broadcasted_iota(jnp.int32, d2.shape, 0)
    big = jnp.float32(jnp.inf)
    for kk in range(k):
        m = jnp.min(d2, axis=0, keepdims=True)
        ii = jnp.min(jnp.where(d2 == m, iota, n_s), axis=0,
                     keepdims=True)
        o_ref[0, kk, :] = ii[0, :]
        d2 = jnp.where(iota == ii, big, d2)


def _knn_idx_pallas(q, s, k, qb):
    b, nq, d = q.shape
    s_n = s.shape[1]
    grid = (b, nq // qb)
    out = pl.pallas_call(
        functools.partial(_knn_kernel, n_s=s_n, k=k),
        grid=grid,
        in_specs=[
            pl.BlockSpec((1, qb, d), lambda i, j: (i, j, 0)),
            pl.BlockSpec((1, s_n, d), lambda i, j: (i, 0, 0)),
        ],
        out_specs=pl.BlockSpec((1, k, qb), lambda i, j: (i, 0, j)),
        out_shape=jax.ShapeDtypeStruct((b, k, nq), jnp.int32),
        compiler_params=pltpu.CompilerParams(
            dimension_semantics=("parallel", "parallel"),
        ),
    )(q, s)
    return jnp.transpose(out, (0, 2, 1))


_GATHER_WINDOW = 128


def _sc_gather(table, flat_idx):
    n_idx = flat_idx.shape[0]
    c = table.shape[1]
    mesh = plsc.VectorSubcoreMesh(core_axis_name="core", subcore_axis_name="subcore")
    idx2 = flat_idx.reshape(1, n_idx)

    @pl.kernel(out_type=jax.ShapeDtypeStruct((n_idx, c), table.dtype), mesh=mesh)
    def gk(x_hbm, i_hbm, o_hbm):
        def body(i_vmem, o_vmem):
            pltpu.sync_copy(x_hbm.at[i_vmem.at[0]], o_vmem)

        pltpu.emit_pipeline(
            body,
            grid=(n_idx // _GATHER_WINDOW,),
            in_specs=[pl.BlockSpec((1, _GATHER_WINDOW), index_map=lambda i: (0, i))],
            out_specs=[pl.BlockSpec((_GATHER_WINDOW, c), index_map=lambda i: (i, 0))],
            core_axis_name="subcore",
            dimension_semantics=(pltpu.PARALLEL,),
        )(i_hbm, o_hbm)

    return gk(table, idx2)


def _conv_block_body(refs, *, c_in, dim, cmid, k, qb, n_w, n_f, n_comb):
    it = iter(refs)
    g_ref = next(it)
    q_ref = next(it)
    w_layers = [(next(it), next(it)) for _ in range(n_w)]
    f_layers = [(next(it), next(it)) for _ in range(n_f)]
    if n_comb:
        x_ref = next(it)
        sp_ref = next(it)
        comb_layers = [(next(it), next(it)) for _ in range(n_comb)]
    out_ref = next(it)

    g3 = g_ref[0].reshape(qb, k, 128)
    q = q_ref[0]
    rel = (g3[:, :, c_in:c_in + dim] - q[:, None, :]).reshape(qb * k, dim)
    h = rel
    for i, (w_r, b_r) in enumerate(w_layers):
        h = jnp.dot(h, w_r[...], preferred_element_type=jnp.float32) + b_r[...]
        if i < n_w - 1:
            h = jax.nn.relu(h)
    w3 = h.reshape(qb, k, cmid)
    f3 = g3[:, :, :c_in]
    m3 = jax.lax.dot_general(w3, f3, (((1,), (1,)), ((0,), (0,))),
                             preferred_element_type=jnp.float32)
    h2 = m3.reshape(qb, cmid * c_in)
    for i, (w_r, b_r) in enumerate(f_layers):
        h2 = jnp.dot(h2, w_r[...], preferred_element_type=jnp.float32) + b_r[...]
        if i < n_f - 1:
            h2 = jax.nn.relu(h2)
    if n_comb:
        cat = jnp.concatenate([x_ref[0], sp_ref[0], h2], axis=1)
        for i, (w_r, b_r) in enumerate(comb_layers):
            cat = jnp.dot(cat, w_r[...], preferred_element_type=jnp.float32) + b_r[...]
            if i < n_comb - 1:
                cat = jax.nn.relu(cat)
        out_ref[0] = cat
    else:
        out_ref[0] = h2


def _conv_pallas(g, q_pts, p, c_in, qb, comb=None, x=None, sp=None):
    b, nq, dim = q_pts.shape
    w_layers, f_layers = p["w"], p["f"]
    cmid = w_layers[-1][0].shape[1]
    comb_layers = comb if comb is not None else []
    c_out = (comb_layers[-1][0].shape[1] if comb_layers else f_layers[-1][0].shape[1])
    grid = (b, nq // qb)

    inputs = [g, q_pts]
    in_specs = [
        pl.BlockSpec((1, qb * K, 128), lambda i, j: (i, j, 0)),
        pl.BlockSpec((1, qb, dim), lambda i, j: (i, j, 0)),
    ]

    def add_weights(layers):
        for w, bb in layers:
            inputs.append(w)
            in_specs.append(pl.BlockSpec(w.shape, lambda i, j: (0, 0)))
            b2 = bb.reshape(1, -1)
            inputs.append(b2)
            in_specs.append(pl.BlockSpec(b2.shape, lambda i, j: (0, 0)))

    add_weights(w_layers)
    add_weights(f_layers)
    if comb_layers:
        inputs.append(x)
        in_specs.append(pl.BlockSpec((1, qb, x.shape[2]), lambda i, j: (i, j, 0)))
        inputs.append(sp)
        in_specs.append(pl.BlockSpec((1, qb, sp.shape[2]), lambda i, j: (i, j, 0)))
        add_weights(comb_layers)

    body = functools.partial(
        _conv_block_body, c_in=c_in, dim=dim, cmid=cmid, k=K, qb=qb,
        n_w=len(w_layers), n_f=len(f_layers), n_comb=len(comb_layers))

    return pl.pallas_call(
        lambda *refs: body(refs),
        grid=grid,
        in_specs=in_specs,
        out_specs=pl.BlockSpec((1, qb, c_out), lambda i, j: (i, j, 0)),
        out_shape=jax.ShapeDtypeStruct((b, nq, c_out), jnp.float32),
        compiler_params=pltpu.CompilerParams(
            dimension_semantics=("parallel", "parallel"),
        ),
    )(*inputs)


def _gather_for_conv(flat_idx, s_pts, feats):
    b, n, c_in = feats.shape
    nq_k = flat_idx.shape[0] // b
    table = jnp.concatenate([feats, s_pts], axis=2).reshape(b * n, c_in + s_pts.shape[2])
    table = jnp.pad(table, ((0, 0), (0, 128 - table.shape[1])))
    return _sc_gather(table, flat_idx).reshape(b, nq_k, 128)


def _flat_idx(idx, n_rows):
    b = idx.shape[0]
    off = (jnp.arange(b, dtype=jnp.int32) * n_rows)[:, None, None]
    return (idx + off).reshape(-1)


_SPLIT = 2


def _conv_split(idx, q_pts, s_pts, feats, p, comb=None, x=None, sp=None):
    b, nq, _ = q_pts.shape
    n = s_pts.shape[1]
    cn = nq // _SPLIT
    table = jnp.concatenate([feats, s_pts], axis=2).reshape(b * n, -1)
    table = jnp.pad(table, ((0, 0), (0, 128 - table.shape[1])))
    outs = []
    for c in range(_SPLIT):
        sl = slice(c * cn, (c + 1) * cn)
        fi = _flat_idx(idx[:, sl, :], n)
        g = _sc_gather(table, fi).reshape(b, cn * K, 128)
        outs.append(_conv_pallas(
            g, q_pts[:, sl], p, feats.shape[2], 512,
            comb=comb,
            x=None if x is None else x[:, sl],
            sp=None if sp is None else sp[:, sl]))
    return jnp.concatenate(outs, axis=1)


def kernel(data, ids, space_pts, time_pts, query_pts, params):
    n = space_pts.shape[1]
    sp_idx = _knn_idx_pallas(space_pts, space_pts, K, 256)
    ti_idx = _knn_idx_pallas(time_pts, time_pts, K, 256)
    q_idx = _knn_idx_pallas(query_pts, time_pts, K, 256)

    x = data
    for i in range(len(params["space"])):
        sp = _conv_split(sp_idx, space_pts, space_pts, x, params["space"][i])
        x = _conv_split(ti_idx, time_pts, time_pts,
                        jnp.concatenate([x, sp], axis=2), params["time"][i],
                        comb=params["comb"][i], x=x, sp=sp)
    g_q = _gather_for_conv(_flat_idx(q_idx, n), time_pts, x)
    return _conv_pallas(g_q, query_pts, params["target"], x.shape[2], 512)


# speedup vs baseline: 1.1040x; 1.1040x over previous
import functools

import jax
import jax.numpy as jnp
from jax.experimental import pallas as pl
from jax.experimental.pallas import tpu as pltpu
from jax.experimental.pallas import tpu_sc as plsc

K = 16


def _mlp(x, layers):
    n = len(layers)
    for i, (W, b) in enumerate(layers):
        x = x @ W + b
        if i < n - 1:
            x = jax.nn.relu(x)
    return x


def _gather(x, idx):
    return jax.vmap(lambda xb, ib: xb[ib])(x, idx)


def _knn_kernel(q_ref, s_ref, o_ref, *, n_s, k):
    q = q_ref[0]
    s = s_ref[0]
    q2 = jnp.sum(q * q, axis=1)
    s2 = jnp.sum(s * s, axis=1)
    qs = jax.lax.dot_general(s, q, (((1,), (1,)), ((), ())),
                             preferred_element_type=jnp.float32)
    d2 = s2[:, None] + q2[None, :] - 2.0 * qs
    iota = jax.lax.broadcasted_iota(jnp.int32, d2.shape, 0)
    big = jnp.float32(jnp.inf)
    for kk in range(k):
        m = jnp.min(d2, axis=0, keepdims=True)
        ii = jnp.min(jnp.where(d2 == m, iota, n_s), axis=0,
                     keepdims=True)
        o_ref[0, kk, :] = ii[0, :]
        d2 = jnp.where(iota == ii, big, d2)


def _knn_idx_pallas(q, s, k, qb):
    b, nq, d = q.shape
    s_n = s.shape[1]
    grid = (b, nq // qb)
    out = pl.pallas_call(
        functools.partial(_knn_kernel, n_s=s_n, k=k),
        grid=grid,
        in_specs=[
            pl.BlockSpec((1, qb, d), lambda i, j: (i, j, 0)),
            pl.BlockSpec((1, s_n, d), lambda i, j: (i, 0, 0)),
        ],
        out_specs=pl.BlockSpec((1, k, qb), lambda i, j: (i, 0, j)),
        out_shape=jax.ShapeDtypeStruct((b, k, nq), jnp.int32),
        compiler_params=pltpu.CompilerParams(
            dimension_semantics=("parallel", "parallel"),
        ),
    )(q, s)
    return jnp.transpose(out, (0, 2, 1))


_GATHER_WINDOW = 128


def _sc_gather(table, flat_idx):
    n_idx = flat_idx.shape[0]
    c = table.shape[1]
    mesh = plsc.VectorSubcoreMesh(core_axis_name="core", subcore_axis_name="subcore")
    idx2 = flat_idx.reshape(1, n_idx)

    half = (n_idx // _GATHER_WINDOW) // 2

    @pl.kernel(out_type=jax.ShapeDtypeStruct((n_idx, c), table.dtype), mesh=mesh)
    def gk(x_hbm, i_hbm, o_hbm):
        cid = jax.lax.axis_index("core")

        def body(i_vmem, o_vmem):
            pltpu.sync_copy(x_hbm.at[i_vmem.at[0]], o_vmem)

        pltpu.emit_pipeline(
            body,
            grid=(half,),
            in_specs=[pl.BlockSpec((1, _GATHER_WINDOW),
                                   index_map=lambda i: (0, i + cid * half))],
            out_specs=[pl.BlockSpec((_GATHER_WINDOW, c),
                                    index_map=lambda i: (i + cid * half, 0))],
            core_axis_name="subcore",
            dimension_semantics=(pltpu.PARALLEL,),
        )(i_hbm, o_hbm)

    return gk(table, idx2)


def _conv_block_body(refs, *, c_in, dim, cmid, k, qb, n_w, n_f, n_comb):
    it = iter(refs)
    g_ref = next(it)
    q_ref = next(it)
    w_layers = [(next(it), next(it)) for _ in range(n_w)]
    f_layers = [(next(it), next(it)) for _ in range(n_f)]
    if n_comb:
        x_ref = next(it)
        sp_ref = next(it)
        comb_layers = [(next(it), next(it)) for _ in range(n_comb)]
    out_ref = next(it)

    g3 = g_ref[0].reshape(qb, k, 128)
    q = q_ref[0]
    rel = (g3[:, :, c_in:c_in + dim] - q[:, None, :]).reshape(qb * k, dim)
    h = rel
    for i, (w_r, b_r) in enumerate(w_layers):
        h = jnp.dot(h, w_r[...], preferred_element_type=jnp.float32) + b_r[...]
        if i < n_w - 1:
            h = jax.nn.relu(h)
    w3 = h.reshape(qb, k, cmid)
    f3 = g3[:, :, :c_in]
    m3 = jax.lax.dot_general(w3, f3, (((1,), (1,)), ((0,), (0,))),
                             preferred_element_type=jnp.float32)
    h2 = m3.reshape(qb, cmid * c_in)
    for i, (w_r, b_r) in enumerate(f_layers):
        h2 = jnp.dot(h2, w_r[...], preferred_element_type=jnp.float32) + b_r[...]
        if i < n_f - 1:
            h2 = jax.nn.relu(h2)
    if n_comb:
        cat = jnp.concatenate([x_ref[0], sp_ref[0], h2], axis=1)
        for i, (w_r, b_r) in enumerate(comb_layers):
            cat = jnp.dot(cat, w_r[...], preferred_element_type=jnp.float32) + b_r[...]
            if i < n_comb - 1:
                cat = jax.nn.relu(cat)
        out_ref[0] = cat
    else:
        out_ref[0] = h2


def _conv_pallas(g, q_pts, p, c_in, qb, comb=None, x=None, sp=None):
    b, nq, dim = q_pts.shape
    w_layers, f_layers = p["w"], p["f"]
    cmid = w_layers[-1][0].shape[1]
    comb_layers = comb if comb is not None else []
    c_out = (comb_layers[-1][0].shape[1] if comb_layers else f_layers[-1][0].shape[1])
    grid = (b, nq // qb)

    inputs = [g, q_pts]
    in_specs = [
        pl.BlockSpec((1, qb * K, 128), lambda i, j: (i, j, 0)),
        pl.BlockSpec((1, qb, dim), lambda i, j: (i, j, 0)),
    ]

    def add_weights(layers):
        for w, bb in layers:
            inputs.append(w)
            in_specs.append(pl.BlockSpec(w.shape, lambda i, j: (0, 0)))
            b2 = bb.reshape(1, -1)
            inputs.append(b2)
            in_specs.append(pl.BlockSpec(b2.shape, lambda i, j: (0, 0)))

    add_weights(w_layers)
    add_weights(f_layers)
    if comb_layers:
        inputs.append(x)
        in_specs.append(pl.BlockSpec((1, qb, x.shape[2]), lambda i, j: (i, j, 0)))
        inputs.append(sp)
        in_specs.append(pl.BlockSpec((1, qb, sp.shape[2]), lambda i, j: (i, j, 0)))
        add_weights(comb_layers)

    body = functools.partial(
        _conv_block_body, c_in=c_in, dim=dim, cmid=cmid, k=K, qb=qb,
        n_w=len(w_layers), n_f=len(f_layers), n_comb=len(comb_layers))

    return pl.pallas_call(
        lambda *refs: body(refs),
        grid=grid,
        in_specs=in_specs,
        out_specs=pl.BlockSpec((1, qb, c_out), lambda i, j: (i, j, 0)),
        out_shape=jax.ShapeDtypeStruct((b, nq, c_out), jnp.float32),
        compiler_params=pltpu.CompilerParams(
            dimension_semantics=("parallel", "parallel"),
        ),
    )(*inputs)


def _gather_for_conv(flat_idx, s_pts, feats):
    b, n, c_in = feats.shape
    nq_k = flat_idx.shape[0] // b
    table = jnp.concatenate([feats, s_pts], axis=2).reshape(b * n, c_in + s_pts.shape[2])
    table = jnp.pad(table, ((0, 0), (0, 128 - table.shape[1])))
    return _sc_gather(table, flat_idx).reshape(b, nq_k, 128)


def _flat_idx(idx, n_rows):
    b = idx.shape[0]
    off = (jnp.arange(b, dtype=jnp.int32) * n_rows)[:, None, None]
    return (idx + off).reshape(-1)


_SPLIT = 2


def _conv_split(idx, q_pts, s_pts, feats, p, comb=None, x=None, sp=None):
    b, nq, _ = q_pts.shape
    n = s_pts.shape[1]
    cn = nq // _SPLIT
    table = jnp.concatenate([feats, s_pts], axis=2).reshape(b * n, -1)
    table = jnp.pad(table, ((0, 0), (0, 128 - table.shape[1])))
    outs = []
    for c in range(_SPLIT):
        sl = slice(c * cn, (c + 1) * cn)
        fi = _flat_idx(idx[:, sl, :], n)
        g = _sc_gather(table, fi).reshape(b, cn * K, 128)
        outs.append(_conv_pallas(
            g, q_pts[:, sl], p, feats.shape[2], 512,
            comb=comb,
            x=None if x is None else x[:, sl],
            sp=None if sp is None else sp[:, sl]))
    return jnp.concatenate(outs, axis=1)


def kernel(data, ids, space_pts, time_pts, query_pts, params):
    n = space_pts.shape[1]
    sp_idx = _knn_idx_pallas(space_pts, space_pts, K, 256)
    ti_idx = _knn_idx_pallas(time_pts, time_pts, K, 256)
    q_idx = _knn_idx_pallas(query_pts, time_pts, K, 256)

    x = data
    for i in range(len(params["space"])):
        sp = _conv_split(sp_idx, space_pts, space_pts, x, params["space"][i])
        x = _conv_split(ti_idx, time_pts, time_pts,
                        jnp.concatenate([x, sp], axis=2), params["time"][i],
                        comb=params["comb"][i], x=x, sp=sp)
    g_q = _gather_for_conv(_flat_idx(q_idx, n), time_pts, x)
    return _conv_pallas(g_q, query_pts, params["target"], x.shape[2], 512)


# speedup vs baseline: 1.1046x; 1.0005x over previous
import functools

import jax
import jax.numpy as jnp
from jax.experimental import pallas as pl
from jax.experimental.pallas import tpu as pltpu
from jax.experimental.pallas import tpu_sc as plsc

K = 16


def _knn_kernel(q_ref, s_ref, o_ref, *, n_s, k):
    q = q_ref[0]
    s = s_ref[0]
    q2 = jnp.sum(q * q, axis=1)
    s2 = jnp.sum(s * s, axis=1)
    qs = jax.lax.dot_general(s, q, (((1,), (1,)), ((), ())),
                             preferred_element_type=jnp.float32)
    d2 = s2[:, None] + q2[None, :] - 2.0 * qs
    iota = jax.lax.broadcasted_iota(jnp.int32, d2.shape, 0)
    big = jnp.float32(jnp.inf)
    for kk in range(k):
        m = jnp.min(d2, axis=0, keepdims=True)
        ii = jnp.min(jnp.where(d2 == m, iota, n_s), axis=0,
                     keepdims=True)
        o_ref[0, kk, :] = ii[0, :]
        d2 = jnp.where(iota == ii, big, d2)


def _knn_idx_pallas(q, s, k, qb):
    b, nq, d = q.shape
    s_n = s.shape[1]
    grid = (b, nq // qb)
    out = pl.pallas_call(
        functools.partial(_knn_kernel, n_s=s_n, k=k),
        grid=grid,
        in_specs=[
            pl.BlockSpec((1, qb, d), lambda i, j: (i, j, 0)),
            pl.BlockSpec((1, s_n, d), lambda i, j: (i, 0, 0)),
        ],
        out_specs=pl.BlockSpec((1, k, qb), lambda i, j: (i, 0, j)),
        out_shape=jax.ShapeDtypeStruct((b, k, nq), jnp.int32),
        compiler_params=pltpu.CompilerParams(
            dimension_semantics=("parallel", "parallel"),
        ),
    )(q, s)
    return jnp.transpose(out, (0, 2, 1))


_GATHER_WINDOW = 128


def _sc_gather(table, flat_idx):
    n_idx = flat_idx.shape[0]
    c = table.shape[1]
    mesh = plsc.VectorSubcoreMesh(core_axis_name="core", subcore_axis_name="subcore")
    idx2 = flat_idx.reshape(1, n_idx)

    half = (n_idx // _GATHER_WINDOW) // 2

    @pl.kernel(out_type=jax.ShapeDtypeStruct((n_idx, c), table.dtype), mesh=mesh)
    def gk(x_hbm, i_hbm, o_hbm):
        cid = jax.lax.axis_index("core")

        def body(i_vmem, o_vmem):
            pltpu.sync_copy(x_hbm.at[i_vmem.at[0]], o_vmem)

        pltpu.emit_pipeline(
            body,
            grid=(half,),
            in_specs=[pl.BlockSpec((1, _GATHER_WINDOW),
                                   index_map=lambda i: (0, i + cid * half))],
            out_specs=[pl.BlockSpec((_GATHER_WINDOW, c),
                                    index_map=lambda i: (i + cid * half, 0))],
            core_axis_name="subcore",
            dimension_semantics=(pltpu.PARALLEL,),
        )(i_hbm, o_hbm)

    return gk(table, idx2)


def _conv_block_body(refs, *, c_in, dim, cmid, k, qb, n_w, n_f, n_comb):
    it = iter(refs)
    g_ref = next(it)
    q_ref = next(it)
    w_layers = [(next(it), next(it)) for _ in range(n_w)]
    f_layers = [(next(it), next(it)) for _ in range(n_f)]
    if n_comb:
        x_ref = next(it)
        sp_ref = next(it)
        comb_layers = [(next(it), next(it)) for _ in range(n_comb)]
    out_ref = next(it)

    g3 = g_ref[0].reshape(qb, k, 128)
    q = q_ref[0]
    rel = (g3[:, :, c_in:c_in + dim] - q[:, None, :]).reshape(qb * k, dim)
    h = rel
    for i, (w_r, b_r) in enumerate(w_layers):
        h = jnp.dot(h, w_r[...], preferred_element_type=jnp.float32) + b_r[...]
        if i < n_w - 1:
            h = jax.nn.relu(h)
    w3 = h.reshape(qb, k, cmid)
    f3 = g3[:, :, :c_in]
    m3 = jax.lax.dot_general(w3, f3, (((1,), (1,)), ((0,), (0,))),
                             preferred_element_type=jnp.float32)
    h2 = m3.reshape(qb, cmid * c_in)
    for i, (w_r, b_r) in enumerate(f_layers):
        h2 = jnp.dot(h2, w_r[...], preferred_element_type=jnp.float32) + b_r[...]
        if i < n_f - 1:
            h2 = jax.nn.relu(h2)
    if n_comb:
        cat = jnp.concatenate([x_ref[0], sp_ref[0], h2], axis=1)
        for i, (w_r, b_r) in enumerate(comb_layers):
            cat = jnp.dot(cat, w_r[...], preferred_element_type=jnp.float32) + b_r[...]
            if i < n_comb - 1:
                cat = jax.nn.relu(cat)
        out_ref[0] = cat
    else:
        out_ref[0] = h2


def _conv_pallas(g, q_pts, p, c_in, qb, comb=None, x=None, sp=None):
    b, nq, dim = q_pts.shape
    w_layers, f_layers = p["w"], p["f"]
    cmid = w_layers[-1][0].shape[1]
    comb_layers = comb if comb is not None else []
    c_out = (comb_layers[-1][0].shape[1] if comb_layers else f_layers[-1][0].shape[1])
    grid = (b, nq // qb)

    inputs = [g, q_pts]
    in_specs = [
        pl.BlockSpec((1, qb * K, 128), lambda i, j: (i, j, 0)),
        pl.BlockSpec((1, qb, dim), lambda i, j: (i, j, 0)),
    ]

    def add_weights(layers):
        for w, bb in layers:
            inputs.append(w)
            in_specs.append(pl.BlockSpec(w.shape, lambda i, j: (0, 0)))
            b2 = bb.reshape(1, -1)
            inputs.append(b2)
            in_specs.append(pl.BlockSpec(b2.shape, lambda i, j: (0, 0)))

    add_weights(w_layers)
    add_weights(f_layers)
    if comb_layers:
        inputs.append(x)
        in_specs.append(pl.BlockSpec((1, qb, x.shape[2]), lambda i, j: (i, j, 0)))
        inputs.append(sp)
        in_specs.append(pl.BlockSpec((1, qb, sp.shape[2]), lambda i, j: (i, j, 0)))
        add_weights(comb_layers)

    body = functools.partial(
        _conv_block_body, c_in=c_in, dim=dim, cmid=cmid, k=K, qb=qb,
        n_w=len(w_layers), n_f=len(f_layers), n_comb=len(comb_layers))

    return pl.pallas_call(
        lambda *refs: body(refs),
        grid=grid,
        in_specs=in_specs,
        out_specs=pl.BlockSpec((1, qb, c_out), lambda i, j: (i, j, 0)),
        out_shape=jax.ShapeDtypeStruct((b, nq, c_out), jnp.float32),
        compiler_params=pltpu.CompilerParams(
            dimension_semantics=("parallel", "parallel"),
        ),
    )(*inputs)


def _gather_for_conv(flat_idx, s_pts, feats):
    b, n, c_in = feats.shape
    nq_k = flat_idx.shape[0] // b
    table = jnp.concatenate([feats, s_pts], axis=2).reshape(b * n, c_in + s_pts.shape[2])
    table = jnp.pad(table, ((0, 0), (0, 128 - table.shape[1])))
    return _sc_gather(table, flat_idx).reshape(b, nq_k, 128)


def _flat_idx(idx, n_rows):
    b = idx.shape[0]
    off = (jnp.arange(b, dtype=jnp.int32) * n_rows)[:, None, None]
    return (idx + off).reshape(-1)


_SPLIT = 2


def _conv_split(idx, q_pts, s_pts, feats, p, comb=None, x=None, sp=None):
    b, nq, _ = q_pts.shape
    n = s_pts.shape[1]
    cn = nq // _SPLIT
    table = jnp.concatenate([feats, s_pts], axis=2).reshape(b * n, -1)
    table = jnp.pad(table, ((0, 0), (0, 128 - table.shape[1])))
    outs = []
    for c in range(_SPLIT):
        sl = slice(c * cn, (c + 1) * cn)
        fi = _flat_idx(idx[:, sl, :], n)
        g = _sc_gather(table, fi).reshape(b, cn * K, 128)
        outs.append(_conv_pallas(
            g, q_pts[:, sl], p, feats.shape[2], 512,
            comb=comb,
            x=None if x is None else x[:, sl],
            sp=None if sp is None else sp[:, sl]))
    return jnp.concatenate(outs, axis=1)


def kernel(data, ids, space_pts, time_pts, query_pts, params):
    n = space_pts.shape[1]
    sp_idx = _knn_idx_pallas(space_pts, space_pts, K, 256)
    ti_idx = _knn_idx_pallas(time_pts, time_pts, K, 256)
    q_idx = _knn_idx_pallas(query_pts, time_pts, K, 256)

    x = data
    for i in range(len(params["space"])):
        sp = _conv_split(sp_idx, space_pts, space_pts, x, params["space"][i])
        x = _conv_split(ti_idx, time_pts, time_pts,
                        jnp.concatenate([x, sp], axis=2), params["time"][i],
                        comb=params["comb"][i], x=x, sp=sp)
    g_q = _gather_for_conv(_flat_idx(q_idx, n), time_pts, x)
    return _conv_pallas(g_q, query_pts, params["target"], x.shape[2], 512)
